# baseline (speedup 1.0000x reference)
import jax
import jax.numpy as jnp
from jax.experimental import pallas as pl
from jax.experimental.pallas import tpu as pltpu

N, D, H = 256, 128, 512
BI = 64
BJ = 64
NI = N // BI
NJ = N // BJ
F32 = jnp.float32


def _mlp(x, w1, b1, w2, b2):
    h = jnp.maximum(jnp.dot(x, w1, preferred_element_type=F32) + b1, 0.0)
    return jnp.dot(h, w2, preferred_element_type=F32) + b2


def _body(bits_ref, i0_ref, i1_ref, i2_ref, i2t_ref,
          w1et_ref, w1eb_ref, b1e_ref, w2e_ref, b2e_ref,
          w1dt_ref, w1db_ref, b1d_ref, w2d_ref, b2d_ref,
          w1g0d_ref, b1g0d_ref, w2g0d_ref, b2g0d_ref,
          w1g0rx_ref, w1g0rn_ref, b1g0r_ref, w2g0r_ref, b2g0r_ref,
          w1g1e_ref, b1g1e_ref, w2g1e_ref, b2g1e_ref,
          w1g1d_ref, b1g1d_ref, w2g1d_ref, b2g1d_ref,
          w1g1rx_ref, w1g1rn_ref, b1g1r_ref, w2g1r_ref, b2g1r_ref,
          out2_ref, out1_ref, out0_ref,
          accmx_ref, accmn_ref, a_ref, bm_ref):
    i = pl.program_id(0)
    j = pl.program_id(1)

    @pl.when((i == 0) & (j == 0))
    def _prep():
        i1v = i1_ref[...]
        a_ref[...] = (jnp.dot(i1v, w1et_ref[...], preferred_element_type=F32)
                      + b1e_ref[...])
        bm_ref[...] = jnp.dot(i1v, w1eb_ref[...], preferred_element_type=F32)

    x = i2_ref[...]
    xt = i2t_ref[...]

    tmx = jnp.max(x, axis=1)
    tmn = jnp.min(x, axis=1)
    rows = pl.ds(i * BI, BI)

    @pl.when(j == 0)
    def _():
        accmx_ref[rows, :] = tmx
        accmn_ref[rows, :] = tmn

    @pl.when(j > 0)
    def _():
        accmx_ref[rows, :] = jnp.maximum(accmx_ref[rows, :], tmx)
        accmn_ref[rows, :] = jnp.minimum(accmn_ref[rows, :], tmn)

    bits5 = bits_ref[5]
    bits6 = bits_ref[6]
    bf16 = jnp.bfloat16
    xr = x.reshape(BI * BJ, D).astype(bf16)
    xtr = jnp.swapaxes(xt, 0, 1).reshape(BI * BJ, D).astype(bf16)
    hd = jnp.maximum(
        jnp.dot(xr, w1dt_ref[...], preferred_element_type=F32)
        + jnp.dot(xtr, w1db_ref[...], preferred_element_type=F32)
        + b1d_ref[...], 0.0).astype(bf16)
    z = bits6 * (jnp.dot(hd, w2d_ref[...], preferred_element_type=F32)
                 + b2d_ref[...])
    a = a_ref[rows, :]
    bm = bm_ref[pl.ds(j * BJ, BJ), :]
    he = jnp.maximum(a[:, None, :] + bm[None, :, :], 0.0) \
        .reshape(BI * BJ, H).astype(bf16)
    z = z + bits5 * (jnp.dot(he, w2e_ref[...], preferred_element_type=F32)
                     + b2e_ref[...])
    out2_ref[...] = jax.nn.sigmoid(z).reshape(BI, BJ, D)

    @pl.when((i == NI - 1) & (j == NJ - 1))
    def _tail():
        i0v = i0_ref[...]
        i1v = i1_ref[...]
        mx1 = jnp.max(i1v, axis=0, keepdims=True)
        mn1 = jnp.min(i1v, axis=0, keepdims=True)
        o0d = _mlp(i0v, w1g0d_ref[...], b1g0d_ref[...],
                   w2g0d_ref[...], b2g0d_ref[...])
        h0r = jnp.maximum(
            jnp.dot(mx1, w1g0rx_ref[...], preferred_element_type=F32)
            + jnp.dot(mn1, w1g0rn_ref[...], preferred_element_type=F32)
            + b1g0r_ref[...], 0.0)
        o0r = jnp.dot(h0r, w2g0r_ref[...], preferred_element_type=F32) \
            + b2g0r_ref[...]
        out0_ref[...] = jax.nn.sigmoid(bits_ref[0] * o0d + bits_ref[1] * o0r)

        e1 = _mlp(i0v, w1g1e_ref[...], b1g1e_ref[...],
                  w2g1e_ref[...], b2g1e_ref[...])
        o1d = _mlp(i1v, w1g1d_ref[...], b1g1d_ref[...],
                   w2g1d_ref[...], b2g1d_ref[...])
        h1r = jnp.maximum(
            jnp.dot(accmx_ref[...], w1g1rx_ref[...], preferred_element_type=F32)
            + jnp.dot(accmn_ref[...], w1g1rn_ref[...], preferred_element_type=F32)
            + b1g1r_ref[...], 0.0)
        o1r = jnp.dot(h1r, w2g1r_ref[...], preferred_element_type=F32) \
            + b2g1r_ref[...]
        out1_ref[...] = jax.nn.sigmoid(
            bits_ref[2] * e1 + bits_ref[3] * o1d + bits_ref[4] * o1r)


def kernel(inputs_0, inputs_1, inputs_2, action, params):
    action = jnp.asarray(action)
    bits = ((action >> (6 - jnp.arange(7, dtype=action.dtype))) & 1).astype(F32)

    i0 = inputs_0.reshape(1, D)
    i1 = inputs_1.reshape(N, D)
    i2 = inputs_2.reshape(N, N, D)

    def row(p):
        return {k: (v.reshape(1, -1) if v.ndim == 1 else v)
                for k, v in p.items()}

    g2e = row(params['g2_exp'])
    g2d = row(params['g2_dir'])
    g0d = row(params['g0_dir'])
    g0r = row(params['g0_red'])
    g1e = row(params['g1_exp'])
    g1d = row(params['g1_dir'])
    g1r = row(params['g1_red'])

    const2 = lambda shape: pl.BlockSpec(shape, lambda i, j: (0, 0))
    const3 = lambda shape: pl.BlockSpec(shape, lambda i, j: (0, 0, 0))

    in_specs = [
        pl.BlockSpec(memory_space=pltpu.SMEM),
        const2((1, D)),
        const2((N, D)),
        pl.BlockSpec((BI, BJ, D), lambda i, j: (i, j, 0)),
        pl.BlockSpec((BJ, BI, D), lambda i, j: (j, i, 0)),
        const2((D, H)), const2((D, H)), const2((1, H)),
        const2((H, D)), const2((1, D)),
        const2((D, H)), const2((D, H)), const2((1, H)),
        const2((H, D)), const2((1, D)),
        const2((D, H)), const2((1, H)), const2((H, D)), const2((1, D)),
        const2((D, H)), const2((D, H)), const2((1, H)),
        const2((H, D)), const2((1, D)),
        const2((D, H)), const2((1, H)), const2((H, D)), const2((1, D)),
        const2((D, H)), const2((1, H)), const2((H, D)), const2((1, D)),
        const2((D, H)), const2((D, H)), const2((1, H)),
        const2((H, D)), const2((1, D)),
    ]
    out_specs = [
        pl.BlockSpec((BI, BJ, D), lambda i, j: (i, j, 0)),
        const2((N, D)),
        const2((1, D)),
    ]
    out_shape = [
        jax.ShapeDtypeStruct((N, N, D), F32),
        jax.ShapeDtypeStruct((N, D), F32),
        jax.ShapeDtypeStruct((1, D), F32),
    ]
    scratch = [
        pltpu.VMEM((N, D), F32),
        pltpu.VMEM((N, D), F32),
        pltpu.VMEM((N, H), F32),
        pltpu.VMEM((N, H), F32),
    ]

    out2, out1, out0 = pl.pallas_call(
        _body,
        grid=(NI, NJ),
        in_specs=in_specs,
        out_specs=out_specs,
        out_shape=out_shape,
        scratch_shapes=scratch,
        compiler_params=pltpu.CompilerParams(
            dimension_semantics=("arbitrary", "arbitrary")),
    )(
        bits, i0, i1, i2, i2,
        g2e['W1'][:D], g2e['W1'][D:], g2e['b1'],
        g2e['W2'].astype(jnp.bfloat16), g2e['b2'],
        g2d['W1'][:D].astype(jnp.bfloat16),
        g2d['W1'][D:].astype(jnp.bfloat16), g2d['b1'],
        g2d['W2'].astype(jnp.bfloat16), g2d['b2'],
        g0d['W1'], g0d['b1'], g0d['W2'], g0d['b2'],
        g0r['W1'][0::2], g0r['W1'][1::2], g0r['b1'], g0r['W2'], g0r['b2'],
        g1e['W1'], g1e['b1'], g1e['W2'], g1e['b2'],
        g1d['W1'], g1d['b1'], g1d['W2'], g1d['b2'],
        g1r['W1'][0::2], g1r['W1'][1::2], g1r['b1'], g1r['W2'], g1r['b2'],
    )

    B = inputs_1.shape[0]
    return (out0.reshape(B, D),
            out1.reshape(B, N, D),
            out2.reshape(B, N, N, D))


# speedup vs baseline: 1.0504x; 1.0504x over previous
import jax
import jax.numpy as jnp
from jax import lax
from jax.experimental import pallas as pl
from jax.experimental.pallas import tpu as pltpu

N, D, H = 256, 128, 512
B_ = 64
NI = N // B_
NP = NI * (NI + 1) // 2
R = B_ * B_
F32 = jnp.float32


def _pair(p):
    i = jnp.int32(0)
    start = 0
    for r in range(1, NI):
        start += NI - (r - 1)
        i = i + (p >= start).astype(jnp.int32)
    base = i * NI - (i * (i - 1)) // 2
    j = p - base + i
    return i, j


def _mlp(x, w1, b1, w2, b2):
    h = jnp.maximum(jnp.dot(x, w1, preferred_element_type=F32) + b1, 0.0)
    return jnp.dot(h, w2, preferred_element_type=F32) + b2


def _body(bits_ref, i0_ref, i1_ref, x_ref, y_ref,
          w1et_ref, w1eb_ref, b1e_ref, w2e_ref, b2e_ref,
          w1dt_ref, w1db_ref, b1d_ref, w2d_ref, b2d_ref,
          w1g0d_ref, b1g0d_ref, w2g0d_ref, b2g0d_ref,
          w1g0rx_ref, w1g0rn_ref, b1g0r_ref, w2g0r_ref, b2g0r_ref,
          w1g1e_ref, b1g1e_ref, w2g1e_ref, b2g1e_ref,
          w1g1d_ref, b1g1d_ref, w2g1d_ref, b2g1d_ref,
          w1g1rx_ref, w1g1rn_ref, b1g1r_ref, w2g1r_ref, b2g1r_ref,
          out2_ref, out1_ref, out0_ref,
          accmx_ref, accmn_ref, a_ref, bm_ref, st_ref, sem):
    p = pl.program_id(0)
    I, J = _pair(p)
    b = lax.rem(p, 2)
    rows_i = pl.ds(I * B_, B_)
    rows_j = pl.ds(J * B_, B_)

    @pl.when(p == 0)
    def _prep():
        i1v = i1_ref[...]
        a_ref[...] = (jnp.dot(i1v, w1et_ref[...], preferred_element_type=F32)
                      + b1e_ref[...])
        bm_ref[...] = jnp.dot(i1v, w1eb_ref[...], preferred_element_type=F32)
        accmx_ref[...] = jnp.full((N, D), -jnp.inf, F32)
        accmn_ref[...] = jnp.full((N, D), jnp.inf, F32)

    x = x_ref[...]
    y = y_ref[...]

    accmx_ref[rows_i, :] = jnp.maximum(accmx_ref[rows_i, :], jnp.max(x, 1))
    accmn_ref[rows_i, :] = jnp.minimum(accmn_ref[rows_i, :], jnp.min(x, 1))

    @pl.when(I != J)
    def _():
        accmx_ref[rows_j, :] = jnp.maximum(accmx_ref[rows_j, :],
                                           jnp.max(y, 1))
        accmn_ref[rows_j, :] = jnp.minimum(accmn_ref[rows_j, :],
                                           jnp.min(y, 1))

    bits5 = bits_ref[5]
    bits6 = bits_ref[6]

    def tile(u, v_t, arows, brows):
        hd = jnp.maximum(
            jnp.dot(u.reshape(R, D), w1dt_ref[...],
                    preferred_element_type=F32)
            + jnp.dot(v_t, w1db_ref[...], preferred_element_type=F32)
            + b1d_ref[...], 0.0)
        z = bits6 * (jnp.dot(hd, w2d_ref[...], preferred_element_type=F32)
                     + b2d_ref[...])
        he = jnp.maximum(a_ref[arows, :][:, None, :]
                         + bm_ref[brows, :][None, :, :], 0.0).reshape(R, H)
        z = z + bits5 * (jnp.dot(he, w2e_ref[...],
                                 preferred_element_type=F32) + b2e_ref[...])
        return jax.nn.sigmoid(z).reshape(B_, B_, D)

    y_t = jnp.swapaxes(y, 0, 1).reshape(R, D)
    st_ref[0, b] = tile(x, y_t, rows_i, rows_j)
    pltpu.make_async_copy(st_ref.at[0, b],
                          out2_ref.at[rows_i, rows_j, :],
                          sem.at[0, b]).start()

    @pl.when(I != J)
    def _():
        x_t = jnp.swapaxes(x, 0, 1).reshape(R, D)
        st_ref[1, b] = tile(y, x_t, rows_j, rows_i)
        pltpu.make_async_copy(st_ref.at[1, b],
                              out2_ref.at[rows_j, rows_i, :],
                              sem.at[1, b]).start()

    @pl.when(p > 0)
    def _():
        q = p - 1
        Iq, Jq = _pair(q)
        bq = lax.rem(q, 2)
        qri = pl.ds(Iq * B_, B_)
        qrj = pl.ds(Jq * B_, B_)
        pltpu.make_async_copy(st_ref.at[0, bq],
                              out2_ref.at[qri, qrj, :],
                              sem.at[0, bq]).wait()

        @pl.when(Iq != Jq)
        def _():
            pltpu.make_async_copy(st_ref.at[1, bq],
                                  out2_ref.at[qrj, qri, :],
                                  sem.at[1, bq]).wait()

    @pl.when(p == NP - 1)
    def _tail():
        pltpu.make_async_copy(st_ref.at[0, b],
                              out2_ref.at[rows_i, rows_j, :],
                              sem.at[0, b]).wait()

        @pl.when(I != J)
        def _():
            pltpu.make_async_copy(st_ref.at[1, b],
                                  out2_ref.at[rows_j, rows_i, :],
                                  sem.at[1, b]).wait()

        i0v = i0_ref[...]
        i1v = i1_ref[...]
        mx1 = jnp.max(i1v, axis=0, keepdims=True)
        mn1 = jnp.min(i1v, axis=0, keepdims=True)
        o0d = _mlp(i0v, w1g0d_ref[...], b1g0d_ref[...],
                   w2g0d_ref[...], b2g0d_ref[...])
        h0r = jnp.maximum(
            jnp.dot(mx1, w1g0rx_ref[...], preferred_element_type=F32)
            + jnp.dot(mn1, w1g0rn_ref[...], preferred_element_type=F32)
            + b1g0r_ref[...], 0.0)
        o0r = jnp.dot(h0r, w2g0r_ref[...], preferred_element_type=F32) \
            + b2g0r_ref[...]
        out0_ref[...] = jax.nn.sigmoid(bits_ref[0] * o0d + bits_ref[1] * o0r)

        e1 = _mlp(i0v, w1g1e_ref[...], b1g1e_ref[...],
                  w2g1e_ref[...], b2g1e_ref[...])
        o1d = _mlp(i1v, w1g1d_ref[...], b1g1d_ref[...],
                   w2g1d_ref[...], b2g1d_ref[...])
        h1r = jnp.maximum(
            jnp.dot(accmx_ref[...], w1g1rx_ref[...],
                    preferred_element_type=F32)
            + jnp.dot(accmn_ref[...], w1g1rn_ref[...],
                      preferred_element_type=F32)
            + b1g1r_ref[...], 0.0)
        o1r = jnp.dot(h1r, w2g1r_ref[...], preferred_element_type=F32) \
            + b2g1r_ref[...]
        out1_ref[...] = jax.nn.sigmoid(
            bits_ref[2] * e1 + bits_ref[3] * o1d + bits_ref[4] * o1r)


def kernel(inputs_0, inputs_1, inputs_2, action, params):
    action = jnp.asarray(action)
    bits = ((action >> (6 - jnp.arange(7, dtype=action.dtype))) & 1).astype(F32)

    i0 = inputs_0.reshape(1, D)
    i1 = inputs_1.reshape(N, D)
    i2 = inputs_2.reshape(N, N, D)

    def row(p):
        return {k: (v.reshape(1, -1) if v.ndim == 1 else v)
                for k, v in p.items()}

    g2e = row(params['g2_exp'])
    g2d = row(params['g2_dir'])
    g0d = row(params['g0_dir'])
    g0r = row(params['g0_red'])
    g1e = row(params['g1_exp'])
    g1d = row(params['g1_dir'])
    g1r = row(params['g1_red'])

    const2 = lambda shape: pl.BlockSpec(shape, lambda p: (0, 0))

    def xmap(p):
        i, j = _pair(p)
        return (i, j, 0)

    def ymap(p):
        i, j = _pair(p)
        return (j, i, 0)

    in_specs = [
        pl.BlockSpec(memory_space=pltpu.SMEM),
        const2((1, D)),
        const2((N, D)),
        pl.BlockSpec((B_, B_, D), xmap),
        pl.BlockSpec((B_, B_, D), ymap),
        const2((D, H)), const2((D, H)), const2((1, H)),
        const2((H, D)), const2((1, D)),
        const2((D, H)), const2((D, H)), const2((1, H)),
        const2((H, D)), const2((1, D)),
        const2((D, H)), const2((1, H)), const2((H, D)), const2((1, D)),
        const2((D, H)), const2((D, H)), const2((1, H)),
        const2((H, D)), const2((1, D)),
        const2((D, H)), const2((1, H)), const2((H, D)), const2((1, D)),
        const2((D, H)), const2((1, H)), const2((H, D)), const2((1, D)),
        const2((D, H)), const2((D, H)), const2((1, H)),
        const2((H, D)), const2((1, D)),
    ]
    out_specs = [
        pl.BlockSpec(memory_space=pl.ANY),
        const2((N, D)),
        const2((1, D)),
    ]
    out_shape = [
        jax.ShapeDtypeStruct((N, N, D), F32),
        jax.ShapeDtypeStruct((N, D), F32),
        jax.ShapeDtypeStruct((1, D), F32),
    ]
    scratch = [
        pltpu.VMEM((N, D), F32),
        pltpu.VMEM((N, D), F32),
        pltpu.VMEM((N, H), F32),
        pltpu.VMEM((N, H), F32),
        pltpu.VMEM((2, 2, B_, B_, D), F32),
        pltpu.SemaphoreType.DMA((2, 2)),
    ]

    out2, out1, out0 = pl.pallas_call(
        _body,
        grid=(NP,),
        in_specs=in_specs,
        out_specs=out_specs,
        out_shape=out_shape,
        scratch_shapes=scratch,
        compiler_params=pltpu.CompilerParams(
            dimension_semantics=("arbitrary",)),
    )(
        bits, i0, i1, i2, i2,
        g2e['W1'][:D], g2e['W1'][D:], g2e['b1'], g2e['W2'], g2e['b2'],
        g2d['W1'][:D], g2d['W1'][D:], g2d['b1'], g2d['W2'], g2d['b2'],
        g0d['W1'], g0d['b1'], g0d['W2'], g0d['b2'],
        g0r['W1'][0::2], g0r['W1'][1::2], g0r['b1'], g0r['W2'], g0r['b2'],
        g1e['W1'], g1e['b1'], g1e['W2'], g1e['b2'],
        g1d['W1'], g1d['b1'], g1d['W2'], g1d['b2'],
        g1r['W1'][0::2], g1r['W1'][1::2], g1r['b1'], g1r['W2'], g1r['b2'],
    )

    B = inputs_1.shape[0]
    return (out0.reshape(B, D),
            out1.reshape(B, N, D),
            out2.reshape(B, N, N, D))
